# baseline (device time: 22676 ns/iter reference)
import jax
import jax.numpy as jnp
from jax import lax
from jax.experimental import pallas as pl
from jax.experimental.pallas import tpu as pltpu

N_DEV = 16
B = 2
SQ = 128
HQ = 8
HKV = 2
DH = 64
D = HQ * DH
SKV = 2048
SKV_PER = SKV // N_DEV
GROUP = HQ // HKV
SCALE = 0.125
ROWS = B * SQ
RPD = ROWS // N_DEV
FL = 384
CLANES = 2 * FL


def kernel(x, Wq, Wo, K_ext, V_ext):
    def body(
        x_ref,
        wq_ref,
        wo_ref,
        k_ref,
        v_ref,
        out_ref,
        obuf,
        orecv,
        agbuf,
        rs_send,
        rs_recv,
        ag_send,
        ag_recv,
    ):
        my = lax.axis_index("i")

        barrier_sem = pltpu.get_barrier_semaphore()
        for d in range(1, N_DEV):
            nbr = lax.rem(my + d, N_DEV)
            pl.semaphore_signal(
                barrier_sem,
                inc=1,
                device_id=(nbr,),
                device_id_type=pl.DeviceIdType.MESH,
            )

        wq = wq_ref[...].astype(jnp.bfloat16)
        wo = wo_ref[...].astype(jnp.bfloat16)
        qs = []
        for b in range(B):
            xb = x_ref[b, :, :].astype(jnp.bfloat16)
            qs.append(jnp.dot(xb, wq, preferred_element_type=jnp.float32))

        sends = []
        recvs = []

        def send_fragment(f):
            orecv[pl.ds(my, 1), :, f * FL : (f + 1) * FL] = obuf[
                :, pl.ds(my * RPD, RPD), f * FL : (f + 1) * FL
            ]
            for d in range(1, N_DEV):
                t = lax.rem(my + d, N_DEV)
                j = N_DEV - 1 - d
                rd = pltpu.make_async_remote_copy(
                    src_ref=obuf.at[:, pl.ds(t * RPD, RPD), f * FL : (f + 1) * FL],
                    dst_ref=orecv.at[pl.ds(my, 1), :, f * FL : (f + 1) * FL],
                    send_sem=rs_send.at[f, d - 1],
                    recv_sem=rs_recv.at[f, j],
                    device_id=(t,),
                    device_id_type=pl.DeviceIdType.MESH,
                )
                rd.start()
                sends.append(rd)
                s_idx = lax.rem(my - d + N_DEV, N_DEV)
                recvs.append(
                    pltpu.make_async_remote_copy(
                        src_ref=orecv.at[pl.ds(s_idx, 1), :, f * FL : (f + 1) * FL],
                        dst_ref=orecv.at[pl.ds(s_idx, 1), :, f * FL : (f + 1) * FL],
                        send_sem=rs_send.at[f, d - 1],
                        recv_sem=rs_recv.at[f, j],
                        device_id=(s_idx,),
                        device_id_type=pl.DeviceIdType.MESH,
                    )
                )

        for hkv in range(HKV):
            for b in range(B):
                qb = qs[b]
                qcat = jnp.concatenate(
                    [
                        qb[:, (hkv * GROUP + j) * DH : (hkv * GROUP + j + 1) * DH]
                        for j in range(GROUP)
                    ],
                    axis=0,
                ).astype(jnp.bfloat16)
                kb = k_ref[b, :, hkv, :].astype(jnp.bfloat16)
                vb = v_ref[b, :, hkv, :].astype(jnp.bfloat16)
                s = (
                    jnp.dot(qcat, kb.T, preferred_element_type=jnp.float32)
                    * SCALE
                )
                p = jnp.exp(s)
                l = jnp.sum(p, axis=1, keepdims=True)
                o = jnp.dot(
                    p.astype(jnp.bfloat16), vb, preferred_element_type=jnp.float32
                )
                o_cat = jnp.concatenate(
                    [o[j * SQ : (j + 1) * SQ] for j in range(GROUP)], axis=1
                ).astype(jnp.bfloat16)
                l_cat = jnp.concatenate(
                    [l[j * SQ : (j + 1) * SQ] for j in range(GROUP)], axis=1
                ).astype(jnp.bfloat16)
                rows = slice(b * SQ, (b + 1) * SQ)
                obuf[0, rows, hkv * FL : hkv * FL + GROUP * DH] = o_cat
                obuf[
                    0, rows, hkv * FL + GROUP * DH : hkv * FL + GROUP * DH + GROUP
                ] = l_cat
            if hkv == 0:
                pl.semaphore_wait(barrier_sem, N_DEV - 1)
            send_fragment(hkv)

        for rdma in recvs:
            rdma.wait_recv()

        o_num = jnp.concatenate(
            [
                jnp.sum(
                    orecv[:, :, f * FL : f * FL + GROUP * DH].astype(
                        jnp.float32
                    ),
                    axis=0,
                )
                for f in range(HKV)
            ],
            axis=1,
        )
        l_x = jnp.concatenate(
            [
                jnp.sum(
                    orecv[
                        :, :, f * FL + GROUP * DH : f * FL + GROUP * DH + GROUP
                    ].astype(jnp.float32),
                    axis=0,
                )
                for f in range(HKV)
            ],
            axis=1,
        )
        head_of_lane = lax.broadcasted_iota(jnp.int32, (HQ, D), 1) // DH
        head_row = lax.broadcasted_iota(jnp.int32, (HQ, D), 0)
        widen = (head_of_lane == head_row).astype(jnp.float32)
        inv_wide = jnp.dot(
            1.0 / l_x, widen, preferred_element_type=jnp.float32
        )
        attn_rows = (o_num * inv_wide).astype(jnp.bfloat16)
        acc = jnp.dot(attn_rows, wo, preferred_element_type=jnp.float32)
        agbuf[pl.ds(my * RPD, RPD), :] = acc.astype(jnp.bfloat16)

        my_out = agbuf.at[pl.ds(my * RPD, RPD), :]
        ag_recvs = []
        for d in range(1, N_DEV):
            t = lax.rem(my + d, N_DEV)
            j = N_DEV - 1 - d
            rd = pltpu.make_async_remote_copy(
                src_ref=my_out,
                dst_ref=my_out,
                send_sem=ag_send.at[d - 1],
                recv_sem=ag_recv.at[j],
                device_id=(t,),
                device_id_type=pl.DeviceIdType.MESH,
            )
            rd.start()
            sends.append(rd)
            s_idx = lax.rem(my - d + N_DEV, N_DEV)
            s_out = agbuf.at[pl.ds(s_idx * RPD, RPD), :]
            ag_recvs.append(
                pltpu.make_async_remote_copy(
                    src_ref=s_out,
                    dst_ref=s_out,
                    send_sem=ag_send.at[d - 1],
                    recv_sem=ag_recv.at[j],
                    device_id=(s_idx,),
                    device_id_type=pl.DeviceIdType.MESH,
                )
            )
        for rdma in ag_recvs:
            rdma.wait_recv()

        for b in range(B):
            out_ref[b, :, :] = agbuf[b * SQ : (b + 1) * SQ, :].astype(
                jnp.float32
            )

        for rdma in sends:
            rdma.wait_send()

    return pl.pallas_call(
        body,
        out_shape=jax.ShapeDtypeStruct((B, SQ, D), jnp.float32),
        in_specs=[pl.BlockSpec(memory_space=pltpu.VMEM)] * 5,
        out_specs=pl.BlockSpec(memory_space=pltpu.VMEM),
        scratch_shapes=[
            pltpu.VMEM((1, ROWS, CLANES), jnp.bfloat16),
            pltpu.VMEM((N_DEV, RPD, CLANES), jnp.bfloat16),
            pltpu.VMEM((ROWS, D), jnp.bfloat16),
            pltpu.SemaphoreType.DMA((HKV, N_DEV - 1)),
            pltpu.SemaphoreType.DMA((HKV, N_DEV - 1)),
            pltpu.SemaphoreType.DMA((N_DEV - 1,)),
            pltpu.SemaphoreType.DMA((N_DEV - 1,)),
        ],
        compiler_params=pltpu.CompilerParams(collective_id=0),
    )(x, Wq, Wo, K_ext, V_ext)


# device time: 21530 ns/iter; 1.0532x vs baseline; 1.0532x over previous
import jax
import jax.numpy as jnp
from jax import lax
from jax.experimental import pallas as pl
from jax.experimental.pallas import tpu as pltpu

N_DEV = 16
B = 2
SQ = 128
HQ = 8
HKV = 2
DH = 64
D = HQ * DH
SKV = 2048
SKV_PER = SKV // N_DEV
GROUP = HQ // HKV
SCALE = 0.125
ROWS = B * SQ
RPD = ROWS // N_DEV
CLANES = 640


def kernel(x, Wq, Wo, K_ext, V_ext):
    def body(
        x_ref,
        wq_ref,
        wo_ref,
        k_ref,
        v_ref,
        out_ref,
        obuf,
        orecv,
        agbuf,
        rs_send,
        rs_recv,
        ag_send,
        ag_recv,
    ):
        my = lax.axis_index("i")

        barrier_sem = pltpu.get_barrier_semaphore()
        for d in range(1, N_DEV):
            nbr = lax.rem(my + d, N_DEV)
            pl.semaphore_signal(
                barrier_sem,
                inc=1,
                device_id=(nbr,),
                device_id_type=pl.DeviceIdType.MESH,
            )

        wq = wq_ref[...].astype(jnp.bfloat16)
        wo = wo_ref[...].astype(jnp.bfloat16)
        for b in range(B):
            xb = x_ref[b, :, :].astype(jnp.bfloat16)
            qb = jnp.dot(xb, wq, preferred_element_type=jnp.float32)
            for hkv in range(HKV):
                qcat = jnp.concatenate(
                    [
                        qb[:, (hkv * GROUP + j) * DH : (hkv * GROUP + j + 1) * DH]
                        for j in range(GROUP)
                    ],
                    axis=0,
                ).astype(jnp.bfloat16)
                kb = k_ref[b, :, hkv, :].astype(jnp.bfloat16)
                vb = v_ref[b, :, hkv, :].astype(jnp.bfloat16)
                s = (
                    jnp.dot(qcat, kb.T, preferred_element_type=jnp.float32)
                    * SCALE
                )
                p = jnp.exp(s)
                l = jnp.sum(p, axis=1, keepdims=True)
                o = jnp.dot(
                    p.astype(jnp.bfloat16), vb, preferred_element_type=jnp.float32
                )
                o_cat = jnp.concatenate(
                    [o[j * SQ : (j + 1) * SQ] for j in range(GROUP)], axis=1
                ).astype(jnp.bfloat16)
                l_cat = jnp.concatenate(
                    [l[j * SQ : (j + 1) * SQ] for j in range(GROUP)], axis=1
                ).astype(jnp.bfloat16)
                rows = slice(b * SQ, (b + 1) * SQ)
                obuf[0, rows, hkv * GROUP * DH : (hkv + 1) * GROUP * DH] = o_cat
                obuf[0, rows, D + hkv * GROUP : D + (hkv + 1) * GROUP] = l_cat

        orecv[pl.ds(my, 1)] = obuf[:, pl.ds(my * RPD, RPD), :]

        pl.semaphore_wait(barrier_sem, N_DEV - 1)

        sends = []
        recvs = []
        for d in range(1, N_DEV):
            t = lax.rem(my + d, N_DEV)
            j = N_DEV - 1 - d
            rd = pltpu.make_async_remote_copy(
                src_ref=obuf.at[:, pl.ds(t * RPD, RPD), :],
                dst_ref=orecv.at[pl.ds(my, 1)],
                send_sem=rs_send.at[d - 1],
                recv_sem=rs_recv.at[j],
                device_id=(t,),
                device_id_type=pl.DeviceIdType.MESH,
            )
            rd.start()
            sends.append(rd)
            s_idx = lax.rem(my - d + N_DEV, N_DEV)
            recvs.append(
                pltpu.make_async_remote_copy(
                    src_ref=orecv.at[pl.ds(s_idx, 1)],
                    dst_ref=orecv.at[pl.ds(s_idx, 1)],
                    send_sem=rs_send.at[d - 1],
                    recv_sem=rs_recv.at[j],
                    device_id=(s_idx,),
                    device_id_type=pl.DeviceIdType.MESH,
                )
            )
        for rdma in recvs:
            rdma.wait_recv()

        o_num = jnp.sum(
            orecv[:, :, 0:D].astype(jnp.float32), axis=0
        )
        l_x = jnp.sum(
            orecv[:, :, D : D + HQ].astype(jnp.float32), axis=0
        )
        head_of_lane = lax.broadcasted_iota(jnp.int32, (HQ, D), 1) // DH
        head_row = lax.broadcasted_iota(jnp.int32, (HQ, D), 0)
        widen = (head_of_lane == head_row).astype(jnp.float32)
        inv_wide = jnp.dot(
            1.0 / l_x, widen, preferred_element_type=jnp.float32
        )
        attn_rows = (o_num * inv_wide).astype(jnp.bfloat16)
        acc = jnp.dot(attn_rows, wo, preferred_element_type=jnp.float32)
        agbuf[pl.ds(my * RPD, RPD), :] = acc.astype(jnp.bfloat16)

        my_out = agbuf.at[pl.ds(my * RPD, RPD), :]
        ag_recvs = []
        for d in range(1, N_DEV):
            t = lax.rem(my + d, N_DEV)
            j = N_DEV - 1 - d
            rd = pltpu.make_async_remote_copy(
                src_ref=my_out,
                dst_ref=my_out,
                send_sem=ag_send.at[d - 1],
                recv_sem=ag_recv.at[j],
                device_id=(t,),
                device_id_type=pl.DeviceIdType.MESH,
            )
            rd.start()
            sends.append(rd)
            s_idx = lax.rem(my - d + N_DEV, N_DEV)
            s_out = agbuf.at[pl.ds(s_idx * RPD, RPD), :]
            ag_recvs.append(
                pltpu.make_async_remote_copy(
                    src_ref=s_out,
                    dst_ref=s_out,
                    send_sem=ag_send.at[d - 1],
                    recv_sem=ag_recv.at[j],
                    device_id=(s_idx,),
                    device_id_type=pl.DeviceIdType.MESH,
                )
            )
        for rdma in ag_recvs:
            rdma.wait_recv()

        for b in range(B):
            out_ref[b, :, :] = agbuf[b * SQ : (b + 1) * SQ, :].astype(
                jnp.float32
            )

        for rdma in sends:
            rdma.wait_send()

    return pl.pallas_call(
        body,
        out_shape=jax.ShapeDtypeStruct((B, SQ, D), jnp.float32),
        in_specs=[pl.BlockSpec(memory_space=pltpu.VMEM)] * 5,
        out_specs=pl.BlockSpec(memory_space=pltpu.VMEM),
        scratch_shapes=[
            pltpu.VMEM((1, ROWS, CLANES), jnp.bfloat16),
            pltpu.VMEM((N_DEV, RPD, CLANES), jnp.bfloat16),
            pltpu.VMEM((ROWS, D), jnp.bfloat16),
            pltpu.SemaphoreType.DMA((N_DEV - 1,)),
            pltpu.SemaphoreType.DMA((N_DEV - 1,)),
            pltpu.SemaphoreType.DMA((N_DEV - 1,)),
            pltpu.SemaphoreType.DMA((N_DEV - 1,)),
        ],
        compiler_params=pltpu.CompilerParams(collective_id=0),
    )(x, Wq, Wo, K_ext, V_ext)
